# baseline (speedup 1.0000x reference)
import functools

import jax
import jax.numpy as jnp
from jax import lax
from jax.experimental import pallas as pl
from jax.experimental.pallas import tpu as pltpu
from jax.experimental.pallas import tpu_sc as plsc

B = 4
T = 32
A = 20000
NC = 2
NS = 16
NW = NC * NS
WPB = NW // B
P = (B * A) // NW
L = 16
PP = ((P + L - 1) // L) * L
CHUNKS = PP // L
G = 8
NPASS = T // G


def _sc_body(pk_hbm, tg_hbm, out_hbm, pk_v, tg_v, st_v, out_v):
    wid = lax.axis_index("s") * NC + lax.axis_index("c")
    b = wid // WPB
    pltpu.sync_copy(pk_hbm.at[wid], pk_v)
    pltpu.sync_copy(tg_hbm.at[b], tg_v)

    lane = lax.iota(jnp.int32, L)

    def make_pass(p):
        first = p == 0
        last = p == NPASS - 1
        tg = [[tg_v[p * G + j, c, :] for c in range(4)] for j in range(G)]
        ta = [tg_v[p * G + j, 4, :] for j in range(G)]

        def body(i, carry):
            s = pl.ds(i * L, L)
            ax1 = pk_v[4, s]
            ay1 = pk_v[5, s]
            ax2 = pk_v[6, s]
            ay2 = pk_v[7, s]
            area_a = jnp.maximum(ax2 - ax1, 0.0) * jnp.maximum(ay2 - ay1, 0.0)
            if first:
                nm = jnp.full((L,), -1.0, jnp.float32)
                dm = jnp.full((L,), 1.0, jnp.float32)
                bc = [jnp.zeros((L,), jnp.float32) for _ in range(4)]
            else:
                nm = st_v[0, s]
                dm = st_v[1, s]
                bc = [st_v[2 + c, s] for c in range(4)]
            for j in range(G):
                tx1, ty1, tx2, ty2 = tg[j]
                iw = jnp.maximum(jnp.minimum(ax2, tx2) - jnp.maximum(ax1, tx1), 0.0)
                ih = jnp.maximum(jnp.minimum(ay2, ty2) - jnp.maximum(ay1, ty1), 0.0)
                inter = iw * ih
                union = jnp.maximum(area_a + ta[j] - inter, 1e-8)
                cond = inter * dm >= nm * union
                nm = jnp.where(cond, inter, nm)
                dm = jnp.where(cond, union, dm)
                bc = [jnp.where(cond, tg[j][c], bc[c]) for c in range(4)]
            if not last:
                st_v[0, s] = nm
                st_v[1, s] = dm
                for c in range(4):
                    st_v[2 + c, s] = bc[c]
                return carry
            acc_r, acc_c, acc_n = carry
            valid = (i * L + lane) < P
            posm = (nm > 0.5 * dm) & valid
            zero = jnp.zeros((L,), jnp.float32)
            for c in range(4):
                y = bc[c] - pk_v[4 + c, s]
                d = jnp.abs(pk_v[c, s] - y)
                elem = jnp.where(d < 1.0, 0.5 * d * d, d - 0.5)
                acc_r = acc_r + jnp.where(posm, elem, zero)
            acc_c = acc_c + jnp.where(posm, pk_v[8, s], zero)
            acc_n = acc_n + jnp.where(posm, jnp.full((L,), 1.0, jnp.float32), zero)
            return acc_r, acc_c, acc_n

        return body

    z = jnp.zeros((L,), jnp.float32)
    carry = (z, z, z)
    for p in range(NPASS):
        carry = lax.fori_loop(0, CHUNKS, make_pass(p), carry)
    acc_r, acc_c, acc_n = carry
    out_v[0, :] = acc_r
    out_v[1, :] = acc_c
    out_v[2, :] = acc_n
    pltpu.sync_copy(out_v, out_hbm.at[wid])


_sc_partials = functools.partial(
    pl.kernel,
    out_type=jax.ShapeDtypeStruct((NW, 3, L), jnp.float32),
    mesh=plsc.VectorSubcoreMesh(core_axis_name="c", subcore_axis_name="s"),
    name="rpn_sc_partials",
    scratch_types=[
        pltpu.VMEM((9, PP), jnp.float32),
        pltpu.VMEM((T, 5, L), jnp.float32),
        pltpu.VMEM((6, PP), jnp.float32),
        pltpu.VMEM((3, L), jnp.float32),
    ],
)(_sc_body)


def _tc_prep_body(reg_ref, cls_ref, anc_ref, tgt_ref, pk_ref, tg_ref, sp_ref):
    pk_ref[:, 0:4, 0:P] = reg_ref[...].transpose(0, 2, 1)
    at = anc_ref[...].transpose(0, 2, 1)
    for b in range(B):
        pk_ref[b * WPB:(b + 1) * WPB, 4:8, 0:P] = at
    pk_ref[:, 8, 0:P] = cls_ref[...]
    t = tgt_ref[...]
    area_t = (jnp.maximum(t[..., 2] - t[..., 0], 0.0)
              * jnp.maximum(t[..., 3] - t[..., 1], 0.0))
    tg = jnp.concatenate([t, area_t[..., None]], axis=-1)
    tg_ref[...] = jnp.broadcast_to(tg[..., None], (B, T, 5, L))
    x = cls_ref[...]
    sp_ref[0, 0] = jnp.sum(jnp.maximum(x, 0.0) + jnp.log1p(jnp.exp(-jnp.abs(x))))


_tc_prep = pl.pallas_call(
    _tc_prep_body,
    out_shape=(
        jax.ShapeDtypeStruct((NW, 9, PP), jnp.float32),
        jax.ShapeDtypeStruct((B, T, 5, L), jnp.float32),
        jax.ShapeDtypeStruct((1, 1), jnp.float32),
    ),
    out_specs=(
        pl.BlockSpec(),
        pl.BlockSpec(),
        pl.BlockSpec(memory_space=pltpu.SMEM),
    ),
)


def kernel(reg, cls, anchors, targets):
    pk, tgp, sp = _tc_prep(reg.reshape(NW, P, 4), cls.reshape(NW, P),
                           anchors.reshape(WPB, P, 4), targets)
    parts = _sc_partials(pk, tgp)

    sums = jnp.sum(parts, axis=(0, 2))
    reg_sum, cls_pos, count = sums[0], sums[1], sums[2]
    reg_loss = jnp.where(count > 0.0,
                         reg_sum / jnp.maximum(count, 1.0), 0.0) * 0.25
    cls_loss = (sp[0, 0] - cls_pos) / jnp.float32(B * A)
    return (jnp.reshape(cls_loss, (1,)), jnp.reshape(reg_loss, (1,)))


# speedup vs baseline: 1.8522x; 1.8522x over previous
import functools

import jax
import jax.numpy as jnp
from jax import lax
from jax.experimental import pallas as pl
from jax.experimental.pallas import tpu as pltpu
from jax.experimental.pallas import tpu_sc as plsc

B = 4
T = 32
A = 20000
NC = 2
NS = 16
NW = NC * NS
WPB = NW // B
P = (B * A) // NW
L = 16
PP = ((P + L - 1) // L) * L
G = 8
NPASS = T // G


def _sc_body(reg_hbm, anc_hbm, cls_hbm, tgte_hbm, areat_hbm,
             out_hbm, reg_v, anc_v, cls_v, tgte_v, areat_v, st_v, out_v, sem):
    wid = lax.axis_index("s") * NC + lax.axis_index("c")
    b = wid // WPB
    cps = [
        pltpu.async_copy(reg_hbm.at[wid], reg_v, sem),
        pltpu.async_copy(anc_hbm.at[wid], anc_v, sem),
        pltpu.async_copy(cls_hbm.at[wid], cls_v, sem),
        pltpu.async_copy(tgte_hbm.at[b], tgte_v, sem),
        pltpu.async_copy(areat_hbm.at[b], areat_v, sem),
    ]
    for cp in cps:
        cp.wait()

    lane = lax.iota(jnp.int32, L)

    def make_pass(p):
        first = p == 0
        last = p == NPASS - 1
        tg = [[tgte_v[p * G + j, c, :] for c in range(4)] for j in range(G)]
        ta = [areat_v[p * G + j, :] for j in range(G)]

        def body(o, carry=None):
            s = pl.ds(o, L)
            ax1 = anc_v[0, s]
            ay1 = anc_v[1, s]
            ax2 = anc_v[2, s]
            ay2 = anc_v[3, s]
            area_a = jnp.maximum(ax2 - ax1, 0.0) * jnp.maximum(ay2 - ay1, 0.0)
            if first:
                nm = jnp.full((L,), -1.0, jnp.float32)
                dm = jnp.full((L,), 1.0, jnp.float32)
                bc = [jnp.zeros((L,), jnp.float32) for _ in range(4)]
            else:
                nm = st_v[0, s]
                dm = st_v[1, s]
                bc = [st_v[2 + c, s] for c in range(4)]
            for j in range(G):
                tx1, ty1, tx2, ty2 = tg[j]
                iw = jnp.maximum(jnp.minimum(ax2, tx2) - jnp.maximum(ax1, tx1), 0.0)
                ih = jnp.maximum(jnp.minimum(ay2, ty2) - jnp.maximum(ay1, ty1), 0.0)
                inter = iw * ih
                union = jnp.maximum(area_a + ta[j] - inter, 1e-8)
                cond = inter * dm >= nm * union
                nm = jnp.where(cond, inter, nm)
                dm = jnp.where(cond, union, dm)
                bc = [jnp.where(cond, tg[j][c], bc[c]) for c in range(4)]
            if not last:
                st_v[0, s] = nm
                st_v[1, s] = dm
                for c in range(4):
                    st_v[2 + c, s] = bc[c]
                return None
            acc_r, acc_c, acc_n = carry
            valid = (o + lane) < P
            posm = (nm > 0.5 * dm) & valid
            zero = jnp.zeros((L,), jnp.float32)
            for c in range(4):
                y = bc[c] - anc_v[c, s]
                d = jnp.abs(reg_v[c, s] - y)
                elem = jnp.where(d < 1.0, 0.5 * d * d, d - 0.5)
                acc_r = acc_r + jnp.where(posm, elem, zero)
            acc_c = acc_c + jnp.where(posm, cls_v[s], zero)
            acc_n = acc_n + jnp.where(posm, jnp.full((L,), 1.0, jnp.float32), zero)
            return acc_r, acc_c, acc_n

        return body

    for p in range(NPASS - 1):
        plsc.parallel_loop(0, PP, L, unroll=2)(
            lambda o, _p=p: make_pass(_p)(o))
    z = jnp.zeros((L,), jnp.float32)
    acc_r, acc_c, acc_n = plsc.parallel_loop(
        0, PP, L, unroll=2, carry=(z, z, z))(make_pass(NPASS - 1))
    out_v[0, :] = acc_r
    out_v[1, :] = acc_c
    out_v[2, :] = acc_n
    pltpu.sync_copy(out_v, out_hbm.at[wid])


_sc_partials = functools.partial(
    pl.kernel,
    out_type=jax.ShapeDtypeStruct((NW, 3, L), jnp.float32),
    mesh=plsc.VectorSubcoreMesh(core_axis_name="c", subcore_axis_name="s"),
    name="rpn_sc_partials",
    scratch_types=[
        pltpu.VMEM((4, PP), jnp.float32),
        pltpu.VMEM((4, PP), jnp.float32),
        pltpu.VMEM((PP,), jnp.float32),
        pltpu.VMEM((T, 4, L), jnp.float32),
        pltpu.VMEM((T, L), jnp.float32),
        pltpu.VMEM((6, PP), jnp.float32),
        pltpu.VMEM((3, L), jnp.float32),
        pltpu.SemaphoreType.DMA,
    ],
)(_sc_body)


def _tc_softplus_body(x_ref, o_ref):
    x = x_ref[...]
    o_ref[0, 0] = jnp.sum(jnp.maximum(x, 0.0) + jnp.log1p(jnp.exp(-jnp.abs(x))))


_tc_softplus = pl.pallas_call(
    _tc_softplus_body,
    out_shape=jax.ShapeDtypeStruct((1, 1), jnp.float32),
    out_specs=pl.BlockSpec(memory_space=pltpu.SMEM),
)


def kernel(reg, cls, anchors, targets):
    reg_r = reg.transpose(0, 2, 1).reshape(B, 4, WPB, P)
    reg_r = reg_r.transpose(0, 2, 1, 3).reshape(NW, 4, P)
    reg_r = jnp.pad(reg_r, ((0, 0), (0, 0), (0, PP - P)))

    anc_r = anchors.transpose(1, 0).reshape(4, WPB, P).transpose(1, 0, 2)
    anc_r = jnp.broadcast_to(anc_r[None], (B, WPB, 4, P)).reshape(NW, 4, P)
    anc_r = jnp.pad(anc_r, ((0, 0), (0, 0), (0, PP - P)))

    cls_r = jnp.pad(cls.reshape(NW, P), ((0, 0), (0, PP - P)))

    tgte = jnp.broadcast_to(targets[..., None], (B, T, 4, L))
    area_t = (jnp.maximum(targets[..., 2] - targets[..., 0], 0.0)
              * jnp.maximum(targets[..., 3] - targets[..., 1], 0.0))
    areat = jnp.broadcast_to(area_t[..., None], (B, T, L))

    parts = _sc_partials(reg_r, anc_r, cls_r, tgte, areat)
    sp = _tc_softplus(cls.reshape(B * A // 128, 128))

    sums = jnp.sum(parts, axis=(0, 2))
    reg_sum, cls_pos, count = sums[0], sums[1], sums[2]
    reg_loss = jnp.where(count > 0.0,
                         reg_sum / jnp.maximum(count, 1.0), 0.0) * 0.25
    cls_loss = (sp[0, 0] - cls_pos) / jnp.float32(B * A)
    return (jnp.reshape(cls_loss, (1,)), jnp.reshape(reg_loss, (1,)))
